# Initial kernel scaffold; baseline (speedup 1.0000x reference)
import jax
import jax.numpy as jnp
from jax.experimental import pallas as pl

ALPHA = -1.0
GROUPS = 1


def kernel(x, embedding):
    bsz, tsz, csz = x.shape
    M = embedding.shape[1]
    emb = embedding[0]
    x_flat = x.reshape(-1, csz)
    distances = (jnp.sum(emb ** 2, axis=1)[None, :]
                 + jnp.sum(x_flat ** 2, axis=1, keepdims=True)
                 - 2.0 * jnp.dot(x_flat, emb.T, precision=jax.lax.Precision.HIGHEST))
    dmap = ALPHA * distances
    dmap = dmap.reshape(bsz * tsz * GROUPS, -1)
    k = jnp.argmax(dmap, axis=-1)
    hard_x = jax.nn.one_hot(k, M, dtype=dmap.dtype).reshape(bsz * tsz, GROUPS, -1)
    hard_probs = jnp.mean(hard_x, axis=0)
    code_perplexity = jnp.squeeze(-jnp.sum(hard_probs * jnp.log2(hard_probs + 1e-10), axis=-1))
    avg_probs = jax.nn.softmax(dmap.reshape(bsz * tsz, GROUPS, -1).astype(jnp.float32), axis=-1).mean(axis=0)
    prob_perplexity = jnp.squeeze(-jnp.sum(avg_probs * jnp.log2(avg_probs + 1e-10), axis=-1))
    dm = hard_x.reshape(bsz * tsz, -1)
    quantization_inds = jnp.argmax(dm.reshape(bsz * tsz * GROUPS, -1), axis=-1).reshape(bsz, tsz, GROUPS)
    quantized = jnp.dot(dm, emb, precision=jax.lax.Precision.HIGHEST).reshape(bsz, tsz, -1)
    return (quantized, code_perplexity, prob_perplexity, quantization_inds)


# speedup vs baseline: 2.1622x; 2.1622x over previous
import functools

import jax
import jax.numpy as jnp
from jax.experimental import pallas as pl
from jax.experimental.pallas import tpu as pltpu

_M = 1024
_D = 256
_BLK = 512


def _vq_kernel(nblocks, n_rows, x_ref, emb_ref, embt_ref,
               q_ref, inds_ref, cp_ref, pp_ref,
               psum_ref, hist_ref):
    i = pl.program_id(0)

    @pl.when(i == 0)
    def _init():
        psum_ref[...] = jnp.zeros_like(psum_ref)
        hist_ref[...] = jnp.zeros_like(hist_ref)

    x = x_ref[...]
    embt = embt_ref[...]
    e2 = jnp.sum(embt * embt, axis=0, keepdims=True)
    x2 = jnp.sum(x * x, axis=1, keepdims=True)
    s = jnp.dot(x.astype(jnp.bfloat16), embt.astype(jnp.bfloat16),
                preferred_element_type=jnp.float32)
    dmap = -((e2 + x2) - 2.0 * s)

    m = jnp.max(dmap, axis=1, keepdims=True)
    iota = jax.lax.broadcasted_iota(jnp.int32, dmap.shape, 1)
    k = jnp.min(jnp.where(dmap == m, iota, _M), axis=1, keepdims=True)
    inds_ref[...] = k

    p = jnp.exp(dmap - m)
    probs = p / jnp.sum(p, axis=1, keepdims=True)
    psum_ref[...] += jnp.sum(probs, axis=0, keepdims=True)

    oh = (iota == k).astype(jnp.float32)
    hist_ref[...] += jnp.sum(oh, axis=0, keepdims=True)

    q_ref[...] = jnp.dot(oh.astype(jnp.bfloat16), emb_ref[...].astype(jnp.bfloat16),
                         preferred_element_type=jnp.float32)

    @pl.when(i == nblocks - 1)
    def _finish():
        inv_n = 1.0 / n_rows
        hp = hist_ref[...] * inv_n
        cp_ref[...] = -jnp.sum(hp * (jnp.log2(hp + 1e-10)), axis=1, keepdims=True)
        ap = psum_ref[...] * inv_n
        pp_ref[...] = -jnp.sum(ap * (jnp.log2(ap + 1e-10)), axis=1, keepdims=True)


def kernel(x, embedding):
    bsz, tsz, csz = x.shape
    n = bsz * tsz
    x_flat = x.reshape(n, csz)
    emb = embedding[0]
    embt = emb.T
    nblocks = n // _BLK

    q, inds, cp, pp = pl.pallas_call(
        functools.partial(_vq_kernel, nblocks, float(n)),
        grid=(nblocks,),
        in_specs=[
            pl.BlockSpec((_BLK, _D), lambda i: (i, 0)),
            pl.BlockSpec((_M, _D), lambda i: (0, 0)),
            pl.BlockSpec((_D, _M), lambda i: (0, 0)),
        ],
        out_specs=[
            pl.BlockSpec((_BLK, _D), lambda i: (i, 0)),
            pl.BlockSpec((_BLK, 1), lambda i: (i, 0)),
            pl.BlockSpec((1, 1), lambda i: (0, 0)),
            pl.BlockSpec((1, 1), lambda i: (0, 0)),
        ],
        out_shape=[
            jax.ShapeDtypeStruct((n, _D), jnp.float32),
            jax.ShapeDtypeStruct((n, 1), jnp.int32),
            jax.ShapeDtypeStruct((1, 1), jnp.float32),
            jax.ShapeDtypeStruct((1, 1), jnp.float32),
        ],
        scratch_shapes=[
            pltpu.VMEM((1, _M), jnp.float32),
            pltpu.VMEM((1, _M), jnp.float32),
        ],
    )(x_flat, emb, embt)

    quantized = q.reshape(bsz, tsz, csz)
    quantization_inds = inds.reshape(bsz, tsz, 1)
    return (quantized, cp[0, 0], pp[0, 0], quantization_inds)
